# Initial kernel scaffold; baseline (speedup 1.0000x reference)
import jax
import jax.numpy as jnp
from jax.experimental import pallas as pl


def kernel(neighbours, shape_features, W1, b1, W2, b2):
    raise NotImplementedError("write your pallas kernel here")


# speedup vs baseline: 1.6677x; 1.6677x over previous
import functools

import jax
import jax.numpy as jnp
from jax import lax
from jax.experimental import pallas as pl
from jax.experimental.pallas import tpu as pltpu
from jax.experimental.pallas import tpu_sc as plsc

_NC = 2
_NS = 16
_NW = _NC * _NS
_LANES = 16


def _aggregate_sc(idx_flat, x, n_pad, K, D, C, T, npw):
    mesh = plsc.VectorSubcoreMesh(core_axis_name="c", subcore_axis_name="s")
    G = C * K

    @functools.partial(
        pl.kernel,
        out_type=jax.ShapeDtypeStruct((n_pad, D), jnp.float32),
        mesh=mesh,
        scratch_types=[
            pltpu.VMEM((G,), jnp.int32),
            pltpu.VMEM((G, D), jnp.float32),
            pltpu.VMEM((C, D), jnp.float32),
            pltpu.SemaphoreType.DMA,
        ],
    )
    def aggr_kernel(idx_hbm, x_hbm, out_hbm, idx_v, rows_v, out_v, sem):
        wid = lax.axis_index("s") * _NC + lax.axis_index("c")

        @pl.loop(0, T)
        def _(t):
            node_base = wid * npw + t * C
            e_base = node_base * K
            pltpu.sync_copy(idx_hbm.at[pl.ds(e_base, G)], idx_v)
            pltpu.async_copy(x_hbm.at[idx_v], rows_v, sem).wait()
            for n in range(C):
                base = n * K
                accs = tuple(
                    rows_v[base, pl.ds(d * _LANES, _LANES)]
                    for d in range(D // _LANES)
                )

                def body(k, a, base=base):
                    return tuple(
                        v + rows_v[base + k, pl.ds(d * _LANES, _LANES)]
                        for d, v in enumerate(a)
                    )

                accs = lax.fori_loop(1, K, body, accs)
                for d, v in enumerate(accs):
                    out_v[n, pl.ds(d * _LANES, _LANES)] = v
            pltpu.sync_copy(out_v, out_hbm.at[pl.ds(node_base, C)])

    return aggr_kernel(idx_flat, x)


def _combine_tc(x, aggr, w1t, w2t, bias, N, D):
    BLK = 1000
    grid = (N // BLK,)

    def body(x_ref, a_ref, w1_ref, w2_ref, b_ref, o_ref):
        acc = jnp.dot(x_ref[...], w1_ref[...],
                      preferred_element_type=jnp.float32,
                      precision=lax.Precision.HIGHEST)
        acc += jnp.dot(a_ref[...], w2_ref[...],
                       preferred_element_type=jnp.float32,
                       precision=lax.Precision.HIGHEST)
        o_ref[...] = jnp.maximum(acc + b_ref[...], 0.0)

    return pl.pallas_call(
        body,
        grid=grid,
        in_specs=[
            pl.BlockSpec((BLK, D), lambda i: (i, 0)),
            pl.BlockSpec((BLK, D), lambda i: (i, 0)),
            pl.BlockSpec((D, D), lambda i: (0, 0)),
            pl.BlockSpec((D, D), lambda i: (0, 0)),
            pl.BlockSpec((1, D), lambda i: (0, 0)),
        ],
        out_specs=pl.BlockSpec((BLK, D), lambda i: (i, 0)),
        out_shape=jax.ShapeDtypeStruct((N, D), jnp.float32),
    )(x, aggr, w1t, w2t, bias)


def kernel(neighbours, shape_features, W1, b1, W2, b2):
    N, K = neighbours.shape
    D = shape_features.shape[1]

    C = 128 // K
    npw = -(-N // _NW)
    npw = -(-npw // C) * C
    T = npw // C
    n_pad = _NW * npw

    nbr_pad = jnp.pad(neighbours, ((0, n_pad - N), (0, 0)))
    idx_flat = nbr_pad.reshape(-1).astype(jnp.int32)

    aggr = _aggregate_sc(idx_flat, shape_features, n_pad, K, D, C, T, npw)

    bias = (b1 + b2).reshape(1, D)
    return _combine_tc(shape_features, aggr[:N], W1.T, W2.T, bias, N, D)
